# Initial kernel scaffold; baseline (speedup 1.0000x reference)
import jax
import jax.numpy as jnp
from jax.experimental import pallas as pl


def kernel(x, Wr1, br1, Wr2, br2, op_logits, op_emb, W1, b1, W2, b2):
    raise NotImplementedError("write your pallas kernel here")


# speedup vs baseline: 1.1305x; 1.1305x over previous
import functools

import jax
import jax.numpy as jnp
from jax import lax
from jax.experimental import pallas as pl
from jax.experimental.pallas import tpu as pltpu
from jax.experimental.pallas import tpu_sc as plsc

_BLK = 128


@functools.lru_cache(maxsize=None)
def _sc_row_gather(n_tbl, n_out, d, chunk):
    info = plsc.get_sparse_core_info()
    nc, ns = info.num_cores, info.num_subcores
    nw = nc * ns
    cpw = n_out // (nw * chunk)
    assert cpw * nw * chunk == n_out, (n_out, nw, chunk)

    mesh = plsc.VectorSubcoreMesh(core_axis_name="c", subcore_axis_name="s")

    @functools.partial(
        pl.kernel,
        mesh=mesh,
        out_type=jax.ShapeDtypeStruct((n_out, d), jnp.float32),
        scratch_types=[
            pltpu.VMEM((cpw, chunk), jnp.int32),
            pltpu.VMEM((chunk, d), jnp.float32),
            pltpu.SemaphoreType.DMA,
        ],
    )
    def gather_kernel(table_hbm, idx_hbm, out_hbm, idx_v, rows_v, sem):
        wid = lax.axis_index("s") * nc + lax.axis_index("c")
        pltpu.sync_copy(idx_hbm.at[pl.ds(wid * cpw, cpw)], idx_v)
        for j in range(cpw):
            pltpu.async_copy(table_hbm.at[idx_v.at[j]], rows_v, sem).wait()
            pltpu.sync_copy(
                rows_v, out_hbm.at[pl.ds((wid * cpw + j) * chunk, chunk)]
            )

    return gather_kernel


def _gather_rows(table, idx2d, n_out, chunk):
    return _sc_row_gather(table.shape[0], n_out, table.shape[1], chunk)(
        table, idx2d
    )


def _c1_body(emb_ref, w1e_ref, b1_ref, o_ref):
    o_ref[0] = (
        jnp.dot(emb_ref[0], w1e_ref[0, 0], preferred_element_type=jnp.float32)
        + b1_ref[0]
    )


def _mlp_body(m_ref, x_ref, w1_ref, c1_ref, w2_ref, b2_ref, o_ref):
    e = w1_ref.shape[2]
    h1 = (
        jnp.dot(x_ref[:, :e], w1_ref[0, 0], preferred_element_type=jnp.float32)
        + jnp.dot(x_ref[:, e:], w1_ref[0, 1], preferred_element_type=jnp.float32)
        + c1_ref[0]
    )
    y = jnp.maximum(h1, 0.0)
    h2 = jnp.dot(y, w2_ref[0], preferred_element_type=jnp.float32)
    o_ref[...] = jnp.maximum(h2 + b2_ref[0], 0.0)


def kernel(x, Wr1, br1, Wr2, br2, op_logits, op_emb, W1, b1, W2, b2):
    n, h = x.shape
    t_n, k_n, e_n = op_emb.shape
    g_n = t_n * k_n
    nblk = n // _BLK + g_n
    p_rows = nblk * _BLK

    key = jax.random.key(42)
    k_route, k_op = jax.random.split(key)
    hmid = jnp.maximum(x @ Wr1 + br1, 0.0)
    route_logits = hmid @ Wr2 + br2
    temper_idx = jax.random.categorical(
        k_route, jax.lax.stop_gradient(route_logits), axis=-1
    )
    t_clip = jnp.clip(temper_idx, 0, t_n - 1)
    tok_op_logits = jnp.take(op_logits, t_clip, axis=0)
    op_idx = jax.random.categorical(
        k_op, jax.lax.stop_gradient(tok_op_logits), axis=-1
    )

    is_halt = temper_idx == t_n
    e_id = jnp.where(is_halt, g_n, t_clip * k_n + op_idx).astype(jnp.int32)

    sort_idx = jnp.argsort(e_id).astype(jnp.int32)
    e_sorted = e_id[sort_idx]
    sizes = jnp.bincount(e_id, length=g_n + 1).astype(jnp.int32)
    bpg = (sizes[:g_n] + _BLK - 1) // _BLK
    blk_start = (jnp.cumsum(bpg) - bpg).astype(jnp.int32)
    grp_off = (jnp.cumsum(sizes) - sizes).astype(jnp.int32)

    slot = jnp.arange(n, dtype=jnp.int32)
    rank = slot - grp_off[e_sorted]
    pos = jnp.where(
        e_sorted < g_n,
        blk_start[jnp.clip(e_sorted, 0, g_n - 1)] * _BLK + rank,
        p_rows,
    ).astype(jnp.int32)
    src_row = jnp.zeros((p_rows,), jnp.int32).at[pos].set(sort_idx, mode="drop")
    pos_tok = (
        jnp.zeros((n,), jnp.int32)
        .at[sort_idx]
        .set(jnp.where(e_sorted < g_n, pos, 0), mode="drop")
    )
    blk = jnp.arange(nblk, dtype=jnp.int32)
    blk_expert = jnp.clip(
        jnp.searchsorted(blk_start, blk, side="right") - 1, 0, g_n - 1
    ).astype(jnp.int32)

    w14 = W1.reshape(g_n, h // e_n + 1, e_n, h)
    w2r = W2.reshape(g_n, h, h)
    emb3 = op_emb.reshape(g_n, 1, e_n)
    b13 = b1.reshape(g_n, 1, h)
    b23 = b2.reshape(g_n, 1, h)

    c13 = pl.pallas_call(
        _c1_body,
        grid=(g_n,),
        in_specs=[
            pl.BlockSpec((1, 1, e_n), lambda g: (g, 0, 0)),
            pl.BlockSpec((1, 1, e_n, h), lambda g: (g, h // e_n, 0, 0)),
            pl.BlockSpec((1, 1, h), lambda g: (g, 0, 0)),
        ],
        out_specs=pl.BlockSpec((1, 1, h), lambda g: (g, 0, 0)),
        out_shape=jax.ShapeDtypeStruct((g_n, 1, h), jnp.float32),
    )(emb3, w14, b13)

    x_pad = _gather_rows(x, src_row.reshape(-1, 80), p_rows, 80)

    grid_spec = pltpu.PrefetchScalarGridSpec(
        num_scalar_prefetch=1,
        grid=(nblk,),
        in_specs=[
            pl.BlockSpec((_BLK, h), lambda i, m: (i, 0)),
            pl.BlockSpec((1, 2, e_n, h), lambda i, m: (m[i], 0, 0, 0)),
            pl.BlockSpec((1, 1, h), lambda i, m: (m[i], 0, 0)),
            pl.BlockSpec((1, h, h), lambda i, m: (m[i], 0, 0)),
            pl.BlockSpec((1, 1, h), lambda i, m: (m[i], 0, 0)),
        ],
        out_specs=pl.BlockSpec((_BLK, h), lambda i, m: (i, 0)),
    )
    y_pad = pl.pallas_call(
        _mlp_body,
        grid_spec=grid_spec,
        out_shape=jax.ShapeDtypeStruct((p_rows, h), jnp.float32),
    )(blk_expert, x_pad, w14, c13, w2r, b23)

    out0 = _gather_rows(y_pad, pos_tok.reshape(-1, 64), n, 64)
    return jnp.where(is_halt[:, None], x, out0)


# speedup vs baseline: 1.7649x; 1.5612x over previous
import functools

import jax
import jax.numpy as jnp
from jax import lax
from jax.experimental import pallas as pl
from jax.experimental.pallas import tpu as pltpu
from jax.experimental.pallas import tpu_sc as plsc

_BLK = 128


@functools.lru_cache(maxsize=None)
def _sc_row_gather(n_tbl, n_out, d, chunk):
    info = plsc.get_sparse_core_info()
    nc, ns = info.num_cores, info.num_subcores
    nw = nc * ns
    cpw = n_out // (nw * chunk)
    assert cpw * nw * chunk == n_out, (n_out, nw, chunk)

    mesh = plsc.VectorSubcoreMesh(core_axis_name="c", subcore_axis_name="s")

    @functools.partial(
        pl.kernel,
        mesh=mesh,
        out_type=jax.ShapeDtypeStruct((n_out, d), jnp.float32),
        scratch_types=[
            pltpu.VMEM((cpw, chunk), jnp.int32),
            pltpu.VMEM((chunk, d), jnp.float32),
            pltpu.SemaphoreType.DMA,
        ],
    )
    def gather_kernel(table_hbm, idx_hbm, out_hbm, idx_v, rows_v, sem):
        wid = lax.axis_index("s") * nc + lax.axis_index("c")
        pltpu.sync_copy(idx_hbm.at[pl.ds(wid * cpw, cpw)], idx_v)
        for j in range(cpw):
            pltpu.async_copy(table_hbm.at[idx_v.at[j]], rows_v, sem).wait()
            pltpu.sync_copy(
                rows_v, out_hbm.at[pl.ds((wid * cpw + j) * chunk, chunk)]
            )

    return gather_kernel


def _gather_rows(table, idx2d, n_out, chunk):
    return _sc_row_gather(table.shape[0], n_out, table.shape[1], chunk)(
        table, idx2d
    )


def _c1_body(emb_ref, w1e_ref, b1_ref, o_ref):
    o_ref[0] = (
        jnp.dot(emb_ref[0], w1e_ref[0, 0], preferred_element_type=jnp.float32)
        + b1_ref[0]
    )


def _mlp_body(m_ref, x_ref, w1_ref, c1_ref, w2_ref, b2_ref, o_ref):
    e = w1_ref.shape[2]
    h1 = (
        jnp.dot(x_ref[:, :e], w1_ref[0, 0], preferred_element_type=jnp.float32)
        + jnp.dot(x_ref[:, e:], w1_ref[0, 1], preferred_element_type=jnp.float32)
        + c1_ref[0]
    )
    y = jnp.maximum(h1, 0.0)
    h2 = jnp.dot(y, w2_ref[0], preferred_element_type=jnp.float32)
    o_ref[...] = jnp.maximum(h2 + b2_ref[0], 0.0)


def kernel(x, Wr1, br1, Wr2, br2, op_logits, op_emb, W1, b1, W2, b2):
    n, h = x.shape
    t_n, k_n, e_n = op_emb.shape
    g_n = t_n * k_n
    nblk = n // _BLK + g_n
    p_rows = nblk * _BLK

    key = jax.random.key(42)
    k_route, k_op = jax.random.split(key)
    hmid = jnp.maximum(x @ Wr1 + br1, 0.0)
    route_logits = hmid @ Wr2 + br2
    temper_idx = jax.random.categorical(
        k_route, jax.lax.stop_gradient(route_logits), axis=-1
    )
    t_clip = jnp.clip(temper_idx, 0, t_n - 1)
    tok_op_logits = jnp.take(op_logits, t_clip, axis=0)
    op_idx = jax.random.categorical(
        k_op, jax.lax.stop_gradient(tok_op_logits), axis=-1
    )

    is_halt = temper_idx == t_n
    e_id = jnp.where(is_halt, g_n, t_clip * k_n + op_idx).astype(jnp.int32)

    sort_idx = jnp.argsort(e_id).astype(jnp.int32)
    e_sorted = e_id[sort_idx]
    sizes = jnp.bincount(e_id, length=g_n + 1).astype(jnp.int32)
    bpg = (sizes[:g_n] + _BLK - 1) // _BLK
    blk_start = (jnp.cumsum(bpg) - bpg).astype(jnp.int32)
    grp_off = (jnp.cumsum(sizes) - sizes).astype(jnp.int32)

    slot = jnp.arange(n, dtype=jnp.int32)
    rank = slot - grp_off[e_sorted]
    pos = jnp.where(
        e_sorted < g_n,
        blk_start[jnp.clip(e_sorted, 0, g_n - 1)] * _BLK + rank,
        p_rows,
    ).astype(jnp.int32)
    pad_fill = (jnp.arange(p_rows, dtype=jnp.int32) % n).astype(jnp.int32)
    src_row = pad_fill.at[pos].set(sort_idx, mode="drop")
    pos_tok = (
        jnp.zeros((n,), jnp.int32)
        .at[sort_idx]
        .set(jnp.where(e_sorted < g_n, pos, 0), mode="drop")
    )
    blk = jnp.arange(nblk, dtype=jnp.int32)
    blk_expert = jnp.clip(
        jnp.searchsorted(blk_start, blk, side="right") - 1, 0, g_n - 1
    ).astype(jnp.int32)

    w14 = W1.reshape(g_n, h // e_n + 1, e_n, h)
    w2r = W2.reshape(g_n, h, h)
    emb3 = op_emb.reshape(g_n, 1, e_n)
    b13 = b1.reshape(g_n, 1, h)
    b23 = b2.reshape(g_n, 1, h)

    c13 = pl.pallas_call(
        _c1_body,
        grid=(g_n,),
        in_specs=[
            pl.BlockSpec((1, 1, e_n), lambda g: (g, 0, 0)),
            pl.BlockSpec((1, 1, e_n, h), lambda g: (g, h // e_n, 0, 0)),
            pl.BlockSpec((1, 1, h), lambda g: (g, 0, 0)),
        ],
        out_specs=pl.BlockSpec((1, 1, h), lambda g: (g, 0, 0)),
        out_shape=jax.ShapeDtypeStruct((g_n, 1, h), jnp.float32),
    )(emb3, w14, b13)

    x_pad = _gather_rows(x, src_row.reshape(-1, 80), p_rows, 80)

    grid_spec = pltpu.PrefetchScalarGridSpec(
        num_scalar_prefetch=1,
        grid=(nblk,),
        in_specs=[
            pl.BlockSpec((_BLK, h), lambda i, m: (i, 0)),
            pl.BlockSpec((1, 2, e_n, h), lambda i, m: (m[i], 0, 0, 0)),
            pl.BlockSpec((1, 1, h), lambda i, m: (m[i], 0, 0)),
            pl.BlockSpec((1, h, h), lambda i, m: (m[i], 0, 0)),
            pl.BlockSpec((1, 1, h), lambda i, m: (m[i], 0, 0)),
        ],
        out_specs=pl.BlockSpec((_BLK, h), lambda i, m: (i, 0)),
    )
    y_pad = pl.pallas_call(
        _mlp_body,
        grid_spec=grid_spec,
        out_shape=jax.ShapeDtypeStruct((p_rows, h), jnp.float32),
    )(blk_expert, x_pad, w14, c13, w2r, b23)

    out0 = _gather_rows(y_pad, pos_tok.reshape(-1, 64), n, 64)
    return jnp.where(is_halt[:, None], x, out0)


# speedup vs baseline: 2.4554x; 1.3912x over previous
import functools

import jax
import jax.numpy as jnp
from jax import lax
from jax.experimental import pallas as pl
from jax.experimental.pallas import tpu as pltpu
from jax.experimental.pallas import tpu_sc as plsc

_BLK = 128


@functools.lru_cache(maxsize=None)
def _sc_row_gather(n_tbl, n_out, d, chunk):
    info = plsc.get_sparse_core_info()
    nc, ns = info.num_cores, info.num_subcores
    nw = nc * ns
    cpw = n_out // (nw * chunk)
    assert cpw * nw * chunk == n_out, (n_out, nw, chunk)
    nbuf = min(cpw, 2)

    mesh = plsc.VectorSubcoreMesh(core_axis_name="c", subcore_axis_name="s")

    @functools.partial(
        pl.kernel,
        mesh=mesh,
        out_type=jax.ShapeDtypeStruct((n_out, d), jnp.float32),
        scratch_types=[
            pltpu.VMEM((cpw, chunk), jnp.int32),
        ]
        + [pltpu.VMEM((chunk, d), jnp.float32) for _ in range(nbuf)]
        + [pltpu.SemaphoreType.DMA for _ in range(nbuf)],
    )
    def gather_kernel(table_hbm, idx_hbm, out_hbm, idx_v, *bufs_sems):
        bufs, sems = bufs_sems[:nbuf], bufs_sems[nbuf:]
        wid = lax.axis_index("s") * nc + lax.axis_index("c")
        pltpu.sync_copy(idx_hbm.at[pl.ds(wid * cpw, cpw)], idx_v)

        def drain(j):
            copies[j].wait()
            pltpu.sync_copy(
                bufs[j % nbuf], out_hbm.at[pl.ds((wid * cpw + j) * chunk, chunk)]
            )

        copies = [None] * cpw
        for j in range(cpw):
            if j >= nbuf:
                drain(j - nbuf)
            copies[j] = pltpu.async_copy(
                table_hbm.at[idx_v.at[j]], bufs[j % nbuf], sems[j % nbuf]
            )
        for j in range(max(0, cpw - nbuf), cpw):
            drain(j)

    return gather_kernel


def _gather_rows(table, idx2d, n_out, chunk):
    return _sc_row_gather(table.shape[0], n_out, table.shape[1], chunk)(
        table, idx2d
    )


def _c1_body(emb_ref, w1e_ref, b1_ref, o_ref):
    o_ref[0] = (
        jnp.dot(emb_ref[0], w1e_ref[0, 0], preferred_element_type=jnp.float32)
        + b1_ref[0]
    )


def _mlp_body(m_ref, md_ref, x_ref, w1_ref, c1_ref, w2_ref, b2_ref, o_ref):
    i = pl.program_id(0)
    e = w1_ref.shape[2]

    @pl.when(md_ref[i] == 0)
    def _matmul():
        h1 = (
            jnp.dot(x_ref[:, :e], w1_ref[0, 0], preferred_element_type=jnp.float32)
            + jnp.dot(x_ref[:, e:], w1_ref[0, 1], preferred_element_type=jnp.float32)
            + c1_ref[0]
        )
        y = jnp.maximum(h1, 0.0)
        h2 = jnp.dot(y, w2_ref[0], preferred_element_type=jnp.float32)
        o_ref[...] = jnp.maximum(h2 + b2_ref[0], 0.0)

    @pl.when(md_ref[i] != 0)
    def _copy():
        o_ref[...] = x_ref[...]


def kernel(x, Wr1, br1, Wr2, br2, op_logits, op_emb, W1, b1, W2, b2):
    n, h = x.shape
    t_n, k_n, e_n = op_emb.shape
    g_n = t_n * k_n
    nblk = n // _BLK + g_n + 8
    p_rows = nblk * _BLK

    key = jax.random.key(42)
    k_route, k_op = jax.random.split(key)
    hmid = jnp.maximum(x @ Wr1 + br1, 0.0)
    route_logits = hmid @ Wr2 + br2
    temper_idx = jax.random.categorical(
        k_route, jax.lax.stop_gradient(route_logits), axis=-1
    )
    t_clip = jnp.clip(temper_idx, 0, t_n - 1)
    tok_op_logits = jnp.take(op_logits, t_clip, axis=0)
    op_idx = jax.random.categorical(
        k_op, jax.lax.stop_gradient(tok_op_logits), axis=-1
    )
    e_id = jnp.where(
        temper_idx == t_n, g_n, t_clip * k_n + op_idx
    ).astype(jnp.int32)

    gids = jnp.arange(g_n + 1, dtype=jnp.int32)
    onehot = (e_id[:, None] == gids[None, :]).astype(jnp.int32)
    counts = jnp.cumsum(onehot, axis=0)
    sizes = counts[-1]
    rank = jnp.sum(onehot * counts, axis=1) - 1
    bpg = (sizes + _BLK - 1) // _BLK
    blk_start = jnp.cumsum(bpg) - bpg
    pos_tok = (
        jnp.sum(onehot * blk_start[None, :], axis=1) * _BLK + rank
    ).astype(jnp.int32)

    pad_fill = (jnp.arange(p_rows, dtype=jnp.int32) % n).astype(jnp.int32)
    src_row = pad_fill.at[pos_tok].set(jnp.arange(n, dtype=jnp.int32))

    blk = jnp.arange(nblk, dtype=jnp.int32)
    blk_expert = jnp.clip(
        jnp.searchsorted(blk_start[:g_n], blk, side="right") - 1, 0, g_n - 1
    ).astype(jnp.int32)
    blk_mode = (blk >= blk_start[g_n]).astype(jnp.int32)

    w14 = W1.reshape(g_n, h // e_n + 1, e_n, h)
    w2r = W2.reshape(g_n, h, h)
    emb3 = op_emb.reshape(g_n, 1, e_n)
    b13 = b1.reshape(g_n, 1, h)
    b23 = b2.reshape(g_n, 1, h)

    c13 = pl.pallas_call(
        _c1_body,
        grid=(g_n,),
        in_specs=[
            pl.BlockSpec((1, 1, e_n), lambda g: (g, 0, 0)),
            pl.BlockSpec((1, 1, e_n, h), lambda g: (g, h // e_n, 0, 0)),
            pl.BlockSpec((1, 1, h), lambda g: (g, 0, 0)),
        ],
        out_specs=pl.BlockSpec((1, 1, h), lambda g: (g, 0, 0)),
        out_shape=jax.ShapeDtypeStruct((g_n, 1, h), jnp.float32),
    )(emb3, w14, b13)

    chunk1 = p_rows // 128
    x_pad = _gather_rows(x, src_row.reshape(-1, chunk1), p_rows, chunk1)

    grid_spec = pltpu.PrefetchScalarGridSpec(
        num_scalar_prefetch=2,
        grid=(nblk,),
        in_specs=[
            pl.BlockSpec((_BLK, h), lambda i, m, md: (i, 0)),
            pl.BlockSpec((1, 2, e_n, h), lambda i, m, md: (m[i], 0, 0, 0)),
            pl.BlockSpec((1, 1, h), lambda i, m, md: (m[i], 0, 0)),
            pl.BlockSpec((1, h, h), lambda i, m, md: (m[i], 0, 0)),
            pl.BlockSpec((1, 1, h), lambda i, m, md: (m[i], 0, 0)),
        ],
        out_specs=pl.BlockSpec((_BLK, h), lambda i, m, md: (i, 0)),
    )
    y_pad = pl.pallas_call(
        _mlp_body,
        grid_spec=grid_spec,
        out_shape=jax.ShapeDtypeStruct((p_rows, h), jnp.float32),
    )(blk_expert, blk_mode, x_pad, w14, c13, w2r, b23)

    return _gather_rows(y_pad, pos_tok.reshape(-1, 64), n, 64)
